# baseline (speedup 1.0000x reference)
import functools

import jax
import jax.numpy as jnp
from jax import lax
from jax.experimental import pallas as pl
from jax.experimental.pallas import tpu as pltpu
from jax.experimental.pallas import tpu_sc as plsc

_B = 4096
_V = 100000
_L = 16

_NC = 2
_NS = 16
_NW = _NC * _NS
_BPW = _B // _NW
_GROUPS = _BPW // _L


def _sc_gather_partials(table, target):
  mesh = plsc.VectorSubcoreMesh(core_axis_name="c", subcore_axis_name="s")

  @functools.partial(
      pl.kernel,
      mesh=mesh,
      out_type=jax.ShapeDtypeStruct((_NW, _L), jnp.float32),
      scratch_types=[
          pltpu.VMEM((_BPW,), jnp.int32),
          pltpu.VMEM((_BPW,), jnp.float32),
          pltpu.VMEM((_L,), jnp.float32),
          pltpu.SemaphoreType.DMA,
      ],
  )
  def sc_kernel(table_hbm, tgt_hbm, out_hbm, idx_v, vals_v, acc_v, sem):
    wid = lax.axis_index("s") * _NC + lax.axis_index("c")
    base = wid * _BPW

    pltpu.sync_copy(tgt_hbm.at[pl.ds(base, _BPW)], idx_v)

    lane_iota = lax.iota(jnp.int32, _L)
    for g in range(_GROUPS):
      t = idx_v[pl.ds(g * _L, _L)]
      pos = (base + g * _L) + lane_iota
      idx_v[pl.ds(g * _L, _L)] = pos * _V + t

    pltpu.async_copy(table_hbm.at[idx_v], vals_v, sem).wait()

    acc = jnp.zeros((_L,), jnp.float32)
    for g in range(_GROUPS):
      acc = acc + vals_v[pl.ds(g * _L, _L)]

    acc_v[...] = acc
    pltpu.sync_copy(acc_v, out_hbm.at[wid])

  return sc_kernel(table, target)


def _tc_reduce(partials):
  def body(x_ref, o_ref):
    o_ref[0, 0] = -jnp.sum(x_ref[...]) * (1.0 / _B)

  return pl.pallas_call(
      body,
      out_shape=jax.ShapeDtypeStruct((1, 1), jnp.float32),
      out_specs=pl.BlockSpec(memory_space=pltpu.SMEM),
  )(partials)


def kernel(input, target):
  table = input.reshape(_B * _V)
  tgt = target.astype(jnp.int32)
  partials = _sc_gather_partials(table, tgt)
  return _tc_reduce(partials)[0, 0]


# speedup vs baseline: 154.6988x; 154.6988x over previous
import functools

import jax
import jax.numpy as jnp
from jax import lax
from jax.experimental import pallas as pl
from jax.experimental.pallas import tpu as pltpu
from jax.experimental.pallas import tpu_sc as plsc

_B = 4096
_V = 100000
_L = 16

_NC = 2
_NS = 16
_NW = _NC * _NS
_BPW = _B // _NW
_GROUPS = _BPW // _L

_ITILES = _B // 128


def _sc_gather_partials(flat, target):
  mesh = plsc.VectorSubcoreMesh(core_axis_name="c", subcore_axis_name="s")

  @functools.partial(
      pl.kernel,
      mesh=mesh,
      out_type=jax.ShapeDtypeStruct((_NW, _L), jnp.float32),
      scratch_types=[
          pltpu.VMEM((_BPW,), jnp.int32),
          pltpu.VMEM((_BPW,), jnp.float32),
          pltpu.VMEM((_L,), jnp.float32),
          pltpu.SemaphoreType.DMA,
      ],
  )
  def sc_kernel(flat_hbm, tgt_hbm, out_hbm, idx_v, vals_v, acc_v, sem):
    wid = lax.axis_index("s") * _NC + lax.axis_index("c")
    base = wid * _BPW

    pltpu.sync_copy(tgt_hbm.at[pl.ds(base, _BPW)], idx_v)

    lane_iota = lax.iota(jnp.int32, _L)
    for g in range(_GROUPS):
      j = idx_v[pl.ds(g * _L, _L)]
      i = (base + g * _L) + lane_iota
      blk = lax.shift_right_logical(j, 3) * _ITILES + lax.shift_right_logical(i, 7)
      sub = lax.shift_left(lax.bitwise_and(j, 7), 7) + lax.bitwise_and(i, 127)
      idx_v[pl.ds(g * _L, _L)] = lax.shift_left(blk, 10) + sub

    pltpu.async_copy(flat_hbm.at[idx_v], vals_v, sem).wait()

    acc = jnp.zeros((_L,), jnp.float32)
    for g in range(_GROUPS):
      acc = acc + vals_v[pl.ds(g * _L, _L)]

    acc_v[...] = acc
    pltpu.sync_copy(acc_v, out_hbm.at[wid])

  return sc_kernel(flat, target)


def _tc_reduce(partials):
  def body(x_ref, o_ref):
    o_ref[0, 0] = -jnp.sum(x_ref[...]) * (1.0 / _B)

  return pl.pallas_call(
      body,
      out_shape=jax.ShapeDtypeStruct((1, 1), jnp.float32),
      out_specs=pl.BlockSpec(memory_space=pltpu.SMEM),
  )(partials)


def kernel(input, target):
  flat = (
      input.T.reshape(_V // 8, 8, _ITILES, 128)
      .transpose(0, 2, 1, 3)
      .reshape(_B * _V)
  )
  tgt = target.astype(jnp.int32)
  partials = _sc_gather_partials(flat, tgt)
  return _tc_reduce(partials)[0, 0]


# speedup vs baseline: 164.8812x; 1.0658x over previous
import functools

import jax
import jax.numpy as jnp
from jax import lax
from jax.experimental import pallas as pl
from jax.experimental.pallas import tpu as pltpu
from jax.experimental.pallas import tpu_sc as plsc

_B = 4096
_V = 100000
_L = 16

_NS = 16
_BPW = _B // _NS
_NCH = _BPW // 128
_GROUPS = 128 // _L

_ITILES = _B // 128


def _sc_gather_partials(flat, target):
  mesh = plsc.VectorSubcoreMesh(
      core_axis_name="c", subcore_axis_name="s", num_cores=1)

  @functools.partial(
      pl.kernel,
      mesh=mesh,
      compiler_params=pltpu.CompilerParams(needs_layout_passes=False),
      out_type=jax.ShapeDtypeStruct((_NS, _L), jnp.float32),
      scratch_types=[
          pltpu.VMEM((_BPW,), jnp.int32),
          pltpu.VMEM((_NCH, 128), jnp.int32),
          pltpu.VMEM((_NCH, 128), jnp.float32),
          pltpu.VMEM((_L,), jnp.float32),
          pltpu.SemaphoreType.DMA,
      ],
  )
  def sc_kernel(flat_hbm, tgt_hbm, out_hbm, tgt_v, idx_v, vals_v, acc_v, sem):
    sid = lax.axis_index("s")
    base = sid * _BPW

    pltpu.sync_copy(tgt_hbm.at[pl.ds(base, _BPW)], tgt_v)

    lane_iota = lax.iota(jnp.int32, _L)
    for k in range(_NCH):
      for g in range(_GROUPS):
        o = k * 128 + g * _L
        j = tgt_v[pl.ds(o, _L)]
        i = (base + o) + lane_iota
        blk = lax.shift_right_logical(j, 3) * _ITILES + lax.shift_right_logical(i, 7)
        sub = lax.shift_left(lax.bitwise_and(j, 7), 7) + lax.bitwise_and(i, 127)
        idx_v[k, pl.ds(g * _L, _L)] = lax.shift_left(blk, 10) + sub

    copies = [
        pltpu.async_copy(flat_hbm.at[idx_v.at[k]], vals_v.at[k], sem)
        for k in range(_NCH)
    ]
    for c in copies:
      c.wait()

    acc = jnp.zeros((_L,), jnp.float32)
    for k in range(_NCH):
      for g in range(_GROUPS):
        acc = acc + vals_v[k, pl.ds(g * _L, _L)]
    acc_v[...] = acc
    pltpu.sync_copy(acc_v, out_hbm.at[sid])

  return sc_kernel(flat, target)


def _tc_reduce(partials):
  def body(x_ref, o_ref):
    o_ref[0, 0] = -jnp.sum(x_ref[...]) * (1.0 / _B)

  return pl.pallas_call(
      body,
      out_shape=jax.ShapeDtypeStruct((1, 1), jnp.float32),
      out_specs=pl.BlockSpec(memory_space=pltpu.SMEM),
  )(partials)


def kernel(input, target):
  flat = (
      input.T.reshape(_V // 8, 8, _ITILES, 128)
      .transpose(0, 2, 1, 3)
      .reshape(_B * _V)
  )
  tgt = target.astype(jnp.int32)
  partials = _sc_gather_partials(flat, tgt)
  return _tc_reduce(partials)[0, 0]


# speedup vs baseline: 168.4400x; 1.0216x over previous
import functools

import jax
import jax.numpy as jnp
from jax import lax
from jax.experimental import pallas as pl
from jax.experimental.pallas import tpu as pltpu
from jax.experimental.pallas import tpu_sc as plsc

_B = 4096
_V = 100000
_L = 16

_NS = 16
_BPW = _B // _NS
_NCH = _BPW // 128
_GROUPS = 128 // _L

_ITILES = _B // 128


def _sc_gather_mean(flat, target):
  mesh = plsc.VectorSubcoreMesh(
      core_axis_name="c", subcore_axis_name="s", num_cores=1)

  @functools.partial(
      pl.kernel,
      mesh=mesh,
      compiler_params=pltpu.CompilerParams(needs_layout_passes=False),
      out_type=(
          jax.ShapeDtypeStruct((_NS, _L), jnp.float32),
          jax.ShapeDtypeStruct((_L,), jnp.float32),
      ),
      scratch_types=[
          pltpu.VMEM((_BPW,), jnp.int32),
          pltpu.VMEM((_NCH, 128), jnp.int32),
          pltpu.VMEM((_NCH, 128), jnp.float32),
          pltpu.VMEM((_L,), jnp.float32),
          pltpu.VMEM((_NS, _L), jnp.float32),
          pltpu.SemaphoreType.DMA,
      ],
  )
  def sc_kernel(flat_hbm, tgt_hbm, stage_hbm, out_hbm,
                tgt_v, idx_v, vals_v, acc_v, all_v, sem):
    sid = lax.axis_index("s")
    base = sid * _BPW

    pltpu.sync_copy(tgt_hbm.at[pl.ds(base, _BPW)], tgt_v)

    lane_iota = lax.iota(jnp.int32, _L)
    for k in range(_NCH):
      for g in range(_GROUPS):
        o = k * 128 + g * _L
        j = tgt_v[pl.ds(o, _L)]
        i = (base + o) + lane_iota
        blk = lax.shift_right_logical(j, 3) * _ITILES + lax.shift_right_logical(i, 7)
        sub = lax.shift_left(lax.bitwise_and(j, 7), 7) + lax.bitwise_and(i, 127)
        idx_v[k, pl.ds(g * _L, _L)] = lax.shift_left(blk, 10) + sub

    copies = [
        pltpu.async_copy(flat_hbm.at[idx_v.at[k]], vals_v.at[k], sem)
        for k in range(_NCH)
    ]
    for c in copies:
      c.wait()

    acc = jnp.zeros((_L,), jnp.float32)
    for k in range(_NCH):
      for g in range(_GROUPS):
        acc = acc + vals_v[k, pl.ds(g * _L, _L)]
    acc_v[...] = acc

    pltpu.sync_copy(acc_v, stage_hbm.at[sid])
    plsc.subcore_barrier()

    @pl.when(sid == 0)
    def _():
      pltpu.sync_copy(stage_hbm, all_v)
      tot = jnp.zeros((_L,), jnp.float32)
      for r in range(_NS):
        tot = tot + all_v[r]
      s = lax.reduce_sum(tot, axes=(0,))
      acc_v[...] = jax.lax.broadcast(s * (-1.0 / _B), (_L,))
      pltpu.sync_copy(acc_v, out_hbm)

  return sc_kernel(flat, target)


def kernel(input, target):
  flat = (
      input.T.reshape(_V // 8, 8, _ITILES, 128)
      .transpose(0, 2, 1, 3)
      .reshape(_B * _V)
  )
  tgt = target.astype(jnp.int32)
  _, final = _sc_gather_mean(flat, tgt)
  return final[0]


# speedup vs baseline: 168.9154x; 1.0028x over previous
import functools

import jax
import jax.numpy as jnp
from jax import lax
from jax.experimental import pallas as pl
from jax.experimental.pallas import tpu as pltpu
from jax.experimental.pallas import tpu_sc as plsc

_B = 4096
_V = 100000
_L = 16

_NS = 16
_BPW = _B // _NS
_NCH = _BPW // 128
_GROUPS = 128 // _L

_ITILES = _B // 128


def _sc_gather_mean(flat, target):
  mesh = plsc.VectorSubcoreMesh(
      core_axis_name="c", subcore_axis_name="s", num_cores=1)

  @functools.partial(
      pl.kernel,
      mesh=mesh,
      compiler_params=pltpu.CompilerParams(needs_layout_passes=False),
      out_type=(
          jax.ShapeDtypeStruct((_NS, _L), jnp.float32),
          jax.ShapeDtypeStruct((_L,), jnp.float32),
      ),
      scratch_types=[
          pltpu.VMEM((_BPW,), jnp.int32),
          pltpu.VMEM((_NCH, 128), jnp.int32),
          pltpu.VMEM((_NCH, 128), jnp.float32),
          pltpu.VMEM((_L,), jnp.float32),
          pltpu.VMEM((_NS, _L), jnp.float32),
          pltpu.SemaphoreType.DMA,
      ],
  )
  def sc_kernel(flat_hbm, tgt_hbm, stage_hbm, out_hbm,
                tgt_v, idx_v, vals_v, acc_v, all_v, sem):
    sid = lax.axis_index("s")
    base = sid * _BPW

    pltpu.sync_copy(tgt_hbm.at[pl.ds(base, _BPW)], tgt_v)

    lane_iota = lax.iota(jnp.int32, _L)

    def idx_body(g, _):
      o = g * _L
      j = tgt_v[pl.ds(o, _L)]
      i = (base + o) + lane_iota
      blk = lax.shift_right_logical(j, 3) * _ITILES + lax.shift_right_logical(i, 7)
      sub = lax.shift_left(lax.bitwise_and(j, 7), 7) + lax.bitwise_and(i, 127)
      idx_v[lax.div(g, _GROUPS), pl.ds(lax.rem(g, _GROUPS) * _L, _L)] = (
          lax.shift_left(blk, 10) + sub)
      return 0

    lax.fori_loop(0, _NCH * _GROUPS, idx_body, 0)

    copies = [
        pltpu.async_copy(flat_hbm.at[idx_v.at[k]], vals_v.at[k], sem)
        for k in range(_NCH)
    ]
    for c in copies:
      c.wait()

    def sum_body(g, acc):
      return acc + vals_v[lax.div(g, _GROUPS), pl.ds(lax.rem(g, _GROUPS) * _L, _L)]

    acc = lax.fori_loop(0, _NCH * _GROUPS, sum_body, jnp.zeros((_L,), jnp.float32))
    acc_v[...] = acc

    pltpu.sync_copy(acc_v, stage_hbm.at[sid])
    plsc.subcore_barrier()

    @pl.when(sid == 0)
    def _():
      pltpu.sync_copy(stage_hbm, all_v)
      tot = jnp.zeros((_L,), jnp.float32)
      for r in range(_NS):
        tot = tot + all_v[r]
      s = lax.reduce_sum(tot, axes=(0,))
      acc_v[...] = jax.lax.broadcast(s * (-1.0 / _B), (_L,))
      pltpu.sync_copy(acc_v, out_hbm)

  return sc_kernel(flat, target)


def kernel(input, target):
  flat = (
      input.T.reshape(_V // 8, 8, _ITILES, 128)
      .transpose(0, 2, 1, 3)
      .reshape(_B * _V)
  )
  tgt = target.astype(jnp.int32)
  _, final = _sc_gather_mean(flat, tgt)
  return final[0]
